# baseline (speedup 1.0000x reference)
import jax
import jax.numpy as jnp
from jax.experimental import pallas as pl
from jax.experimental.pallas import tpu as pltpu

_ROWS, _COLS = 1048576, 64


def _patch_kernel(x_ref, o_ref, vbuf, sem_in, sem_out):
    del x_ref
    cp_in = pltpu.make_async_copy(o_ref.at[pl.ds(0, 8), :], vbuf, sem_in)
    cp_in.start()
    cp_in.wait()
    vbuf[2:3, :] = jnp.full((1, _COLS), 9.0, jnp.float32)
    cp_out = pltpu.make_async_copy(vbuf, o_ref.at[pl.ds(0, 8), :], sem_out)
    cp_out.start()
    cp_out.wait()


def kernel(x):
    return pl.pallas_call(
        _patch_kernel,
        in_specs=[pl.BlockSpec(memory_space=pl.ANY)],
        out_specs=pl.BlockSpec(memory_space=pl.ANY),
        out_shape=jax.ShapeDtypeStruct((_ROWS, _COLS), jnp.float32),
        input_output_aliases={0: 0},
        scratch_shapes=[
            pltpu.VMEM((8, _COLS), jnp.float32),
            pltpu.SemaphoreType.DMA,
            pltpu.SemaphoreType.DMA,
        ],
    )(x)


# speedup vs baseline: 1.0014x; 1.0014x over previous
import jax
import jax.numpy as jnp
from jax.experimental import pallas as pl

_ROWS, _COLS = 1048576, 64


def _patch_kernel(x_ref, o_ref):
    blk = x_ref[...]
    row = jax.lax.broadcasted_iota(jnp.int32, (8, _COLS), 0)
    o_ref[...] = jnp.where(row == 2, jnp.float32(9.0), blk)


def kernel(x):
    return pl.pallas_call(
        _patch_kernel,
        grid=(1,),
        in_specs=[pl.BlockSpec((8, _COLS), lambda i: (0, 0))],
        out_specs=pl.BlockSpec((8, _COLS), lambda i: (0, 0)),
        out_shape=jax.ShapeDtypeStruct((_ROWS, _COLS), jnp.float32),
        input_output_aliases={0: 0},
    )(x)


# speedup vs baseline: 4.1919x; 4.1862x over previous
import jax
import jax.numpy as jnp
from jax.experimental import pallas as pl

_ROWS, _COLS = 1048576, 64


def _patch_kernel(x_ref, o_ref):
    blk = x_ref[...]
    col = jax.lax.broadcasted_iota(jnp.int32, (8, 128), 1)
    o_ref[...] = jnp.where(col == 2, jnp.float32(9.0), blk)


def kernel(x):
    xt = x.T
    out_t = pl.pallas_call(
        _patch_kernel,
        grid=(_COLS // 8,),
        in_specs=[pl.BlockSpec((8, 128), lambda i: (i, 0))],
        out_specs=pl.BlockSpec((8, 128), lambda i: (i, 0)),
        out_shape=jax.ShapeDtypeStruct((_COLS, _ROWS), jnp.float32),
        input_output_aliases={0: 0},
    )(xt)
    return out_t.T


# speedup vs baseline: 4.2549x; 1.0150x over previous
import jax
import jax.numpy as jnp
from jax.experimental import pallas as pl

_ROWS, _COLS = 1048576, 64


def _patch_kernel(x_ref, o_ref):
    blk = x_ref[...]
    col = jax.lax.broadcasted_iota(jnp.int32, (_COLS, 128), 1)
    o_ref[...] = jnp.where(col == 2, jnp.float32(9.0), blk)


def kernel(x):
    xt = x.T
    out_t = pl.pallas_call(
        _patch_kernel,
        grid=(1,),
        in_specs=[pl.BlockSpec((_COLS, 128), lambda i: (0, 0))],
        out_specs=pl.BlockSpec((_COLS, 128), lambda i: (0, 0)),
        out_shape=jax.ShapeDtypeStruct((_COLS, _ROWS), jnp.float32),
        input_output_aliases={0: 0},
    )(xt)
    return out_t.T
